# Initial kernel scaffold; baseline (speedup 1.0000x reference)
import jax
import jax.numpy as jnp
from jax.experimental import pallas as pl


def kernel(x, W_enc, b_enc, W_dec, b_dec):
    raise NotImplementedError("write your pallas kernel here")


# speedup vs baseline: 8.5066x; 8.5066x over previous
import functools

import jax
import jax.numpy as jnp
from jax.experimental import pallas as pl
from jax.experimental.pallas import tpu as pltpu

N_TOK = 4096
D_ACT = 2048
D_DICT = 16384
TOPK = 64

M_BLK = 128
N_TILE = 2048
M2_BLK = 1024
K2_TILE = 2048


def _encode_select_body(x_ref, wenc_ref, benc_ref, bdec_ref, z_ref, pre_ref):
    n = pl.program_id(1)
    n_last = pl.num_programs(1) - 1

    xb = (x_ref[...] - bdec_ref[...]).astype(jnp.bfloat16)
    acc = jax.lax.dot_general(
        xb, wenc_ref[...], (((1,), (1,)), ((), ())),
        preferred_element_type=jnp.float32)
    pre = jnp.maximum(acc + benc_ref[...], 0.0)
    pre_ref[:, pl.ds(n * N_TILE, N_TILE)] = pre

    @pl.when(n == n_last)
    def _select():
        pre_all = pre_ref[...]
        bits = jax.lax.bitcast_convert_type(pre_all, jnp.int32)

        def step(i, lo):
            cand = lo | (jnp.int32(1) << (jnp.int32(30) - i))
            cnt = jnp.sum((bits >= cand).astype(jnp.int32), axis=1,
                          keepdims=True)
            return jnp.where(cnt >= TOPK, cand, lo)

        lo = jax.lax.fori_loop(
            0, 31, step, jnp.zeros((M_BLK, 1), jnp.int32))
        thr = jax.lax.bitcast_convert_type(lo, jnp.float32)
        z_ref[...] = jnp.where(pre_all >= thr, pre_all, 0.0)


def _decode_body(z_ref, wdec_ref, bdec_ref, out_ref):
    k = pl.program_id(1)
    zb = z_ref[...].astype(jnp.bfloat16)
    part = jax.lax.dot_general(
        zb, wdec_ref[...], (((1,), (1,)), ((), ())),
        preferred_element_type=jnp.float32)

    @pl.when(k == 0)
    def _init():
        out_ref[...] = part + bdec_ref[...]

    @pl.when(k != 0)
    def _acc():
        out_ref[...] += part


@jax.jit
def kernel(x, W_enc, b_enc, W_dec, b_dec):
    wenc_bf = W_enc.astype(jnp.bfloat16)
    wdec_bf = W_dec.astype(jnp.bfloat16)
    benc2 = b_enc.reshape(1, D_DICT)
    bdec2 = b_dec.reshape(1, D_ACT)

    sparse_z = pl.pallas_call(
        _encode_select_body,
        grid=(N_TOK // M_BLK, D_DICT // N_TILE),
        in_specs=[
            pl.BlockSpec((M_BLK, D_ACT), lambda m, n: (m, 0)),
            pl.BlockSpec((N_TILE, D_ACT), lambda m, n: (n, 0)),
            pl.BlockSpec((1, N_TILE), lambda m, n: (0, n)),
            pl.BlockSpec((1, D_ACT), lambda m, n: (0, 0)),
        ],
        out_specs=pl.BlockSpec((M_BLK, D_DICT), lambda m, n: (m, 0)),
        out_shape=jax.ShapeDtypeStruct((N_TOK, D_DICT), jnp.float32),
        scratch_shapes=[pltpu.VMEM((M_BLK, D_DICT), jnp.float32)],
        compiler_params=pltpu.CompilerParams(
            dimension_semantics=("arbitrary", "arbitrary")),
    )(x, wenc_bf, benc2, bdec2)

    x_hat = pl.pallas_call(
        _decode_body,
        grid=(N_TOK // M2_BLK, D_DICT // K2_TILE),
        in_specs=[
            pl.BlockSpec((M2_BLK, K2_TILE), lambda m, k: (m, k)),
            pl.BlockSpec((D_ACT, K2_TILE), lambda m, k: (0, k)),
            pl.BlockSpec((1, D_ACT), lambda m, k: (0, 0)),
        ],
        out_specs=pl.BlockSpec((M2_BLK, D_ACT), lambda m, k: (m, 0)),
        out_shape=jax.ShapeDtypeStruct((N_TOK, D_ACT), jnp.float32),
        compiler_params=pltpu.CompilerParams(
            dimension_semantics=("arbitrary", "arbitrary")),
    )(sparse_z, wdec_bf, bdec2)

    return (x_hat, sparse_z)


# speedup vs baseline: 9.5315x; 1.1205x over previous
import jax
import jax.numpy as jnp
from jax.experimental import pallas as pl
from jax.experimental.pallas import tpu as pltpu

N_TOK = 4096
D_ACT = 2048
D_DICT = 16384
TOPK = 64

M1_BLK = 1024
N1_TILE = 2048
MS_BLK = 256
S_CHUNK = 2048
M3_BLK = 512
K3_TILE = 2048


def _encode_body(x_ref, wenc_ref, benc_ref, bdec_ref, pre_ref):
    xb = (x_ref[...] - bdec_ref[...]).astype(jnp.bfloat16)
    acc = jax.lax.dot_general(
        xb, wenc_ref[...], (((1,), (1,)), ((), ())),
        preferred_element_type=jnp.float32)
    pre_ref[...] = jnp.maximum(acc + benc_ref[...], 0.0)


def _select_body(pre_ref, thr_ref):
    def step(i, lo):
        cand = lo | (jnp.int32(1) << (jnp.int32(30) - i))
        candf = jax.lax.bitcast_convert_type(cand, jnp.float32)

        def chunk(c, acc):
            tile = pre_ref[:, pl.ds(c * S_CHUNK, S_CHUNK)]
            return acc + jnp.sum((tile >= candf).astype(jnp.int32),
                                 axis=1, keepdims=True)

        cnt = jax.lax.fori_loop(0, D_DICT // S_CHUNK, chunk,
                                jnp.zeros((MS_BLK, 1), jnp.int32))
        return jnp.where(cnt >= TOPK, cand, lo)

    lo = jax.lax.fori_loop(0, 31, step, jnp.zeros((MS_BLK, 1), jnp.int32))
    thr = jax.lax.bitcast_convert_type(lo, jnp.float32)
    thr_ref[...] = jnp.broadcast_to(thr, (MS_BLK, 128))


def _decode_body(pre_ref, thr_ref, wdec_ref, bdec_ref, out_ref, z_ref):
    k = pl.program_id(1)
    pre = pre_ref[...]
    z = jnp.where(pre >= thr_ref[...][:, 0:1], pre, 0.0)
    z_ref[...] = z
    part = jax.lax.dot_general(
        z.astype(jnp.bfloat16), wdec_ref[...], (((1,), (1,)), ((), ())),
        preferred_element_type=jnp.float32)

    @pl.when(k == 0)
    def _init():
        out_ref[...] = part + bdec_ref[...]

    @pl.when(k != 0)
    def _acc():
        out_ref[...] += part


@jax.jit
def kernel(x, W_enc, b_enc, W_dec, b_dec):
    wenc_bf = W_enc.astype(jnp.bfloat16)
    wdec_bf = W_dec.astype(jnp.bfloat16)
    benc2 = b_enc.reshape(1, D_DICT)
    bdec2 = b_dec.reshape(1, D_ACT)

    pre = pl.pallas_call(
        _encode_body,
        grid=(N_TOK // M1_BLK, D_DICT // N1_TILE),
        in_specs=[
            pl.BlockSpec((M1_BLK, D_ACT), lambda m, n: (m, 0)),
            pl.BlockSpec((N1_TILE, D_ACT), lambda m, n: (n, 0)),
            pl.BlockSpec((1, N1_TILE), lambda m, n: (0, n)),
            pl.BlockSpec((1, D_ACT), lambda m, n: (0, 0)),
        ],
        out_specs=pl.BlockSpec((M1_BLK, N1_TILE), lambda m, n: (m, n)),
        out_shape=jax.ShapeDtypeStruct((N_TOK, D_DICT), jnp.float32),
        compiler_params=pltpu.CompilerParams(
            dimension_semantics=("arbitrary", "arbitrary")),
    )(x, wenc_bf, benc2, bdec2)

    thr = pl.pallas_call(
        _select_body,
        grid=(N_TOK // MS_BLK,),
        in_specs=[pl.BlockSpec((MS_BLK, D_DICT), lambda m: (m, 0))],
        out_specs=pl.BlockSpec((MS_BLK, 128), lambda m: (m, 0)),
        out_shape=jax.ShapeDtypeStruct((N_TOK, 128), jnp.float32),
        compiler_params=pltpu.CompilerParams(
            dimension_semantics=("arbitrary",)),
    )(pre)

    x_hat, sparse_z = pl.pallas_call(
        _decode_body,
        grid=(N_TOK // M3_BLK, D_DICT // K3_TILE),
        in_specs=[
            pl.BlockSpec((M3_BLK, K3_TILE), lambda m, k: (m, k)),
            pl.BlockSpec((M3_BLK, 128), lambda m, k: (m, 0)),
            pl.BlockSpec((D_ACT, K3_TILE), lambda m, k: (0, k)),
            pl.BlockSpec((1, D_ACT), lambda m, k: (0, 0)),
        ],
        out_specs=(
            pl.BlockSpec((M3_BLK, D_ACT), lambda m, k: (m, 0)),
            pl.BlockSpec((M3_BLK, K3_TILE), lambda m, k: (m, k)),
        ),
        out_shape=(
            jax.ShapeDtypeStruct((N_TOK, D_ACT), jnp.float32),
            jax.ShapeDtypeStruct((N_TOK, D_DICT), jnp.float32),
        ),
        compiler_params=pltpu.CompilerParams(
            dimension_semantics=("arbitrary", "arbitrary")),
    )(pre, thr, wdec_bf, bdec2)

    return (x_hat, sparse_z)


# speedup vs baseline: 10.6588x; 1.1183x over previous
import jax
import jax.numpy as jnp
from jax.experimental import pallas as pl
from jax.experimental.pallas import tpu as pltpu

N_TOK = 4096
D_ACT = 2048
D_DICT = 16384
TOPK = 64

M1_BLK = 1024
N1_TILE = 2048
MS_BLK = 256
S_CHUNK = 2048
M3_BLK = 512
K3_TILE = 2048


def _encode_body(x_ref, wenc_ref, benc_ref, bdec_ref, pre_ref):
    xb = (x_ref[...] - bdec_ref[...]).astype(jnp.bfloat16)
    acc = jax.lax.dot_general(
        xb, wenc_ref[...], (((1,), (1,)), ((), ())),
        preferred_element_type=jnp.float32)
    pre_ref[...] = jnp.maximum(acc + benc_ref[...], 0.0)


def _select_body(pre_ref, thr_ref, tr_ref):
    one_b = jnp.bfloat16(1.0)
    zero_b = jnp.bfloat16(0.0)

    def build(c, carry):
        tile = pre_ref[:, pl.ds(c * S_CHUNK, S_CHUNK)]
        bits = jax.lax.bitcast_convert_type(tile, jnp.int32)
        tr = jax.lax.bitcast_convert_type(
            bits & jnp.int32(-65536), jnp.float32)
        tr_ref[:, pl.ds(c * S_CHUNK, S_CHUNK)] = tr.astype(jnp.bfloat16)
        return carry

    jax.lax.fori_loop(0, D_DICT // S_CHUNK, build, jnp.int32(0))

    def step_a(i, lo):
        cand = lo | (jnp.int32(1) << (jnp.int32(30) - i))
        candf = jax.lax.bitcast_convert_type(cand, jnp.float32)
        candb = candf.astype(jnp.bfloat16)

        def chunk(c, acc):
            tile = tr_ref[:, pl.ds(c * S_CHUNK, S_CHUNK)]
            m = jnp.where(tile >= candb, one_b, zero_b)
            h = m
            w = S_CHUNK
            while w > 128:
                h = h[:, :w // 2] + h[:, w // 2:]
                w //= 2
            return acc + jnp.sum(h.astype(jnp.float32), axis=1,
                                 keepdims=True)

        cnt = jax.lax.fori_loop(0, D_DICT // S_CHUNK, chunk,
                                jnp.zeros((MS_BLK, 1), jnp.float32))
        return jnp.where(cnt >= TOPK, cand, lo)

    def step_b(i, lo):
        cand = lo | (jnp.int32(1) << (jnp.int32(30) - i))
        candf = jax.lax.bitcast_convert_type(cand, jnp.float32)

        def chunk(c, acc):
            tile = pre_ref[:, pl.ds(c * S_CHUNK, S_CHUNK)]
            return acc + jnp.sum((tile >= candf).astype(jnp.int32),
                                 axis=1, keepdims=True)

        cnt = jax.lax.fori_loop(0, D_DICT // S_CHUNK, chunk,
                                jnp.zeros((MS_BLK, 1), jnp.int32))
        return jnp.where(cnt >= TOPK, cand, lo)

    lo = jax.lax.fori_loop(0, 15, step_a, jnp.zeros((MS_BLK, 1), jnp.int32))
    lo = jax.lax.fori_loop(15, 31, step_b, lo)
    thr = jax.lax.bitcast_convert_type(lo, jnp.float32)
    thr_ref[...] = jnp.broadcast_to(thr, (MS_BLK, 128))


def _decode_body(pre_ref, thr_ref, wdec_ref, bdec_ref, out_ref, z_ref):
    k = pl.program_id(1)
    pre = pre_ref[...]
    z = jnp.where(pre >= thr_ref[...][:, 0:1], pre, 0.0)
    z_ref[...] = z
    part = jax.lax.dot_general(
        z.astype(jnp.bfloat16), wdec_ref[...], (((1,), (1,)), ((), ())),
        preferred_element_type=jnp.float32)

    @pl.when(k == 0)
    def _init():
        out_ref[...] = part + bdec_ref[...]

    @pl.when(k != 0)
    def _acc():
        out_ref[...] += part


@jax.jit
def kernel(x, W_enc, b_enc, W_dec, b_dec):
    wenc_bf = W_enc.astype(jnp.bfloat16)
    wdec_bf = W_dec.astype(jnp.bfloat16)
    benc2 = b_enc.reshape(1, D_DICT)
    bdec2 = b_dec.reshape(1, D_ACT)

    pre = pl.pallas_call(
        _encode_body,
        grid=(N_TOK // M1_BLK, D_DICT // N1_TILE),
        in_specs=[
            pl.BlockSpec((M1_BLK, D_ACT), lambda m, n: (m, 0)),
            pl.BlockSpec((N1_TILE, D_ACT), lambda m, n: (n, 0)),
            pl.BlockSpec((1, N1_TILE), lambda m, n: (0, n)),
            pl.BlockSpec((1, D_ACT), lambda m, n: (0, 0)),
        ],
        out_specs=pl.BlockSpec((M1_BLK, N1_TILE), lambda m, n: (m, n)),
        out_shape=jax.ShapeDtypeStruct((N_TOK, D_DICT), jnp.float32),
        compiler_params=pltpu.CompilerParams(
            dimension_semantics=("arbitrary", "arbitrary")),
    )(x, wenc_bf, benc2, bdec2)

    thr = pl.pallas_call(
        _select_body,
        grid=(N_TOK // MS_BLK,),
        in_specs=[pl.BlockSpec((MS_BLK, D_DICT), lambda m: (m, 0))],
        out_specs=pl.BlockSpec((MS_BLK, 128), lambda m: (m, 0)),
        out_shape=jax.ShapeDtypeStruct((N_TOK, 128), jnp.float32),
        scratch_shapes=[pltpu.VMEM((MS_BLK, D_DICT), jnp.bfloat16)],
        compiler_params=pltpu.CompilerParams(
            dimension_semantics=("arbitrary",)),
    )(pre)

    x_hat, sparse_z = pl.pallas_call(
        _decode_body,
        grid=(N_TOK // M3_BLK, D_DICT // K3_TILE),
        in_specs=[
            pl.BlockSpec((M3_BLK, K3_TILE), lambda m, k: (m, k)),
            pl.BlockSpec((M3_BLK, 128), lambda m, k: (m, 0)),
            pl.BlockSpec((D_ACT, K3_TILE), lambda m, k: (0, k)),
            pl.BlockSpec((1, D_ACT), lambda m, k: (0, 0)),
        ],
        out_specs=(
            pl.BlockSpec((M3_BLK, D_ACT), lambda m, k: (m, 0)),
            pl.BlockSpec((M3_BLK, K3_TILE), lambda m, k: (m, k)),
        ),
        out_shape=(
            jax.ShapeDtypeStruct((N_TOK, D_ACT), jnp.float32),
            jax.ShapeDtypeStruct((N_TOK, D_DICT), jnp.float32),
        ),
        compiler_params=pltpu.CompilerParams(
            dimension_semantics=("arbitrary", "arbitrary")),
    )(pre, thr, wdec_bf, bdec2)

    return (x_hat, sparse_z)


# speedup vs baseline: 11.1571x; 1.0468x over previous
import jax
import jax.numpy as jnp
from jax.experimental import pallas as pl
from jax.experimental.pallas import tpu as pltpu

N_TOK = 4096
D_ACT = 2048
D_DICT = 16384
TOPK = 64

M1_BLK = 1024
N1_TILE = 2048
MS_BLK = 256
S_CHUNK = 2048
M3_BLK = 512
K3_TILE = 2048


def _encode_body(x_ref, wenc_ref, benc_ref, bdec_ref, pre_ref):
    xb = (x_ref[...] - bdec_ref[...]).astype(jnp.bfloat16)
    acc = jax.lax.dot_general(
        xb, wenc_ref[...], (((1,), (1,)), ((), ())),
        preferred_element_type=jnp.float32)
    pre_ref[...] = jnp.maximum(acc + benc_ref[...], 0.0)


def _select_body(pre_ref, thr_ref, tr_ref, k16_ref):
    one_b = jnp.bfloat16(1.0)
    zero_b = jnp.bfloat16(0.0)

    def build(c, carry):
        tile = pre_ref[:, pl.ds(c * S_CHUNK, S_CHUNK)]
        bits = jax.lax.bitcast_convert_type(tile, jnp.int32)
        tr = jax.lax.bitcast_convert_type(
            bits & jnp.int32(-65536), jnp.float32)
        tr_ref[:, pl.ds(c * S_CHUNK, S_CHUNK)] = tr.astype(jnp.bfloat16)
        return carry

    jax.lax.fori_loop(0, D_DICT // S_CHUNK, build, jnp.int32(0))

    def step_a(i, lo):
        cand = lo | (jnp.int32(1) << (jnp.int32(30) - i))
        candf = jax.lax.bitcast_convert_type(cand, jnp.float32)
        candb = candf.astype(jnp.bfloat16)

        def chunk(c, acc):
            tile = tr_ref[:, pl.ds(c * S_CHUNK, S_CHUNK)]
            m = jnp.where(tile >= candb, one_b, zero_b)
            h = m
            w = S_CHUNK
            while w > 128:
                h = h[:, :w // 2] + h[:, w // 2:]
                w //= 2
            return acc + jnp.sum(h.astype(jnp.float32), axis=1,
                                 keepdims=True)

        cnt = jax.lax.fori_loop(0, D_DICT // S_CHUNK, chunk,
                                jnp.zeros((MS_BLK, 1), jnp.float32))
        return jnp.where(cnt >= TOPK, cand, lo)

    T = jax.lax.fori_loop(0, 15, step_a, jnp.zeros((MS_BLK, 1), jnp.int32))

    def build_b(c, cnt_hi):
        tile = pre_ref[:, pl.ds(c * S_CHUNK, S_CHUNK)]
        bits = jax.lax.bitcast_convert_type(tile, jnp.int32)
        eq = (bits & jnp.int32(-65536)) == T
        key = jnp.where(eq, (bits & jnp.int32(65535)) - 32768,
                        jnp.int32(-32768)).astype(jnp.int16)
        k16_ref[:, pl.ds(c * S_CHUNK, S_CHUNK)] = key
        hi = jnp.sum((bits >= T + 65536).astype(jnp.int32), axis=1,
                     keepdims=True)
        return cnt_hi + hi

    cnt_hi = jax.lax.fori_loop(0, D_DICT // S_CHUNK, build_b,
                               jnp.zeros((MS_BLK, 1), jnp.int32))

    one_s = jnp.int16(1)
    zero_s = jnp.int16(0)

    def step_c(i, u_lo):
        u_cand = u_lo | (jnp.int32(1) << (jnp.int32(15) - i))
        cand_s = (u_cand - 32768).astype(jnp.int16)

        def chunk(c, acc):
            tile = k16_ref[:, pl.ds(c * S_CHUNK, S_CHUNK)]
            m = jnp.where(tile >= cand_s, one_s, zero_s)
            h = m
            w = S_CHUNK
            while w > 128:
                h = h[:, :w // 2] + h[:, w // 2:]
                w //= 2
            return acc + jnp.sum(h.astype(jnp.int32), axis=1,
                                 keepdims=True)

        cnt = cnt_hi + jax.lax.fori_loop(0, D_DICT // S_CHUNK, chunk,
                                         jnp.zeros((MS_BLK, 1), jnp.int32))
        return jnp.where(cnt >= TOPK, u_cand, u_lo)

    u = jax.lax.fori_loop(0, 16, step_c, jnp.zeros((MS_BLK, 1), jnp.int32))
    thr = jax.lax.bitcast_convert_type(T | u, jnp.float32)
    thr_ref[...] = jnp.broadcast_to(thr, (MS_BLK, 128))


def _decode_body(pre_ref, thr_ref, wdec_ref, bdec_ref, out_ref, z_ref):
    k = pl.program_id(1)
    pre = pre_ref[...]
    z = jnp.where(pre >= thr_ref[...][:, 0:1], pre, 0.0)
    z_ref[...] = z
    part = jax.lax.dot_general(
        z.astype(jnp.bfloat16), wdec_ref[...], (((1,), (1,)), ((), ())),
        preferred_element_type=jnp.float32)

    @pl.when(k == 0)
    def _init():
        out_ref[...] = part + bdec_ref[...]

    @pl.when(k != 0)
    def _acc():
        out_ref[...] += part


@jax.jit
def kernel(x, W_enc, b_enc, W_dec, b_dec):
    wenc_bf = W_enc.astype(jnp.bfloat16)
    wdec_bf = W_dec.astype(jnp.bfloat16)
    benc2 = b_enc.reshape(1, D_DICT)
    bdec2 = b_dec.reshape(1, D_ACT)

    pre = pl.pallas_call(
        _encode_body,
        grid=(N_TOK // M1_BLK, D_DICT // N1_TILE),
        in_specs=[
            pl.BlockSpec((M1_BLK, D_ACT), lambda m, n: (m, 0)),
            pl.BlockSpec((N1_TILE, D_ACT), lambda m, n: (n, 0)),
            pl.BlockSpec((1, N1_TILE), lambda m, n: (0, n)),
            pl.BlockSpec((1, D_ACT), lambda m, n: (0, 0)),
        ],
        out_specs=pl.BlockSpec((M1_BLK, N1_TILE), lambda m, n: (m, n)),
        out_shape=jax.ShapeDtypeStruct((N_TOK, D_DICT), jnp.float32),
        compiler_params=pltpu.CompilerParams(
            dimension_semantics=("arbitrary", "arbitrary")),
    )(x, wenc_bf, benc2, bdec2)

    thr = pl.pallas_call(
        _select_body,
        grid=(N_TOK // MS_BLK,),
        in_specs=[pl.BlockSpec((MS_BLK, D_DICT), lambda m: (m, 0))],
        out_specs=pl.BlockSpec((MS_BLK, 128), lambda m: (m, 0)),
        out_shape=jax.ShapeDtypeStruct((N_TOK, 128), jnp.float32),
        scratch_shapes=[pltpu.VMEM((MS_BLK, D_DICT), jnp.bfloat16),
                        pltpu.VMEM((MS_BLK, D_DICT), jnp.int16)],
        compiler_params=pltpu.CompilerParams(
            dimension_semantics=("arbitrary",)),
    )(pre)

    x_hat, sparse_z = pl.pallas_call(
        _decode_body,
        grid=(N_TOK // M3_BLK, D_DICT // K3_TILE),
        in_specs=[
            pl.BlockSpec((M3_BLK, K3_TILE), lambda m, k: (m, k)),
            pl.BlockSpec((M3_BLK, 128), lambda m, k: (m, 0)),
            pl.BlockSpec((D_ACT, K3_TILE), lambda m, k: (0, k)),
            pl.BlockSpec((1, D_ACT), lambda m, k: (0, 0)),
        ],
        out_specs=(
            pl.BlockSpec((M3_BLK, D_ACT), lambda m, k: (m, 0)),
            pl.BlockSpec((M3_BLK, K3_TILE), lambda m, k: (m, k)),
        ),
        out_shape=(
            jax.ShapeDtypeStruct((N_TOK, D_ACT), jnp.float32),
            jax.ShapeDtypeStruct((N_TOK, D_DICT), jnp.float32),
        ),
        compiler_params=pltpu.CompilerParams(
            dimension_semantics=("arbitrary", "arbitrary")),
    )(pre, thr, wdec_bf, bdec2)

    return (x_hat, sparse_z)
